# baseline (speedup 1.0000x reference)
import functools

import jax
import jax.numpy as jnp
import numpy as np
from jax import lax
from jax.experimental import pallas as pl
from jax.experimental.pallas import tpu as pltpu
from jax.experimental.pallas import tpu_sc as plsc

_MODEL_DIM = 1024
_MAX_POS = 1200
_TEMP = 10000.0
_D = _MODEL_DIM // 2
_H = _D // 2

_NC, _NS, _L = 2, 16, 16
_NW = _NC * _NS
_T = 16

_SC_FRAC_NUM, _SC_FRAC_DEN = 1, 2
_BT = 256


@functools.lru_cache(maxsize=1)
def _fused_table():
    positions = np.arange(_MAX_POS, dtype=np.float64)[:, None]
    div_term = np.exp(np.arange(0, _D, 2, dtype=np.float64) * -(np.log(_TEMP) / _D))
    ang = positions * div_term
    tab = np.concatenate([np.cos(ang), np.sin(ang)], axis=-1)
    return tab.astype(np.float32)


@functools.lru_cache(maxsize=1)
def _div_term():
    d = np.exp(np.arange(0, _D, 2, dtype=np.float64) * -(np.log(_TEMP) / _D))
    return d.astype(np.float32).reshape(1, _H)


def _make_sc_rope(n_tokens: int, n_sc: int):
    per_w = n_sc // _NW
    n_chunks = per_w // _T
    assert n_chunks % 2 == 0 and n_chunks >= 4
    mesh = plsc.VectorSubcoreMesh(core_axis_name="c", subcore_axis_name="s")

    @functools.partial(
        pl.kernel,
        mesh=mesh,
        out_type=jax.ShapeDtypeStruct((n_tokens, _MODEL_DIM), jnp.float32),
        scratch_types=[
            pltpu.VMEM((2 * per_w,), jnp.int32),
            pltpu.VMEM((2 * _T, _D), jnp.float32),
            pltpu.VMEM((2 * _T, _D), jnp.float32),
            pltpu.VMEM((_T, _MODEL_DIM), jnp.float32),
            pltpu.VMEM((_T, _MODEL_DIM), jnp.float32),
            pltpu.SemaphoreType.DMA,
            pltpu.SemaphoreType.DMA,
            pltpu.SemaphoreType.DMA,
            pltpu.SemaphoreType.DMA,
            pltpu.SemaphoreType.DMA,
            pltpu.SemaphoreType.DMA,
        ],
    )
    def sc_rope(
        tab_hbm, x_hbm, pos_hbm, out_hbm,
        idx_all, rows0, rows1, x0, x1,
        gs0, gs1, xs0, xs1, os0, os1,
    ):
        rows = (rows0, rows1)
        xbuf = (x0, x1)
        gsem = (gs0, gs1)
        xsem = (xs0, xs1)
        osem = (os0, os1)
        wid = lax.axis_index("s") * _NC + lax.axis_index("c")
        tok0 = wid * per_w

        pltpu.sync_copy(pos_hbm.at[pl.ds(2 * tok0, 2 * per_w)], idx_all)

        def fetch(ci, b):
            pltpu.async_copy(
                tab_hbm.at[idx_all.at[pl.ds(ci * 2 * _T, 2 * _T)]], rows[b], gsem[b]
            )
            pltpu.async_copy(x_hbm.at[pl.ds(tok0 + ci * _T, _T)], xbuf[b], xsem[b])

        def wait_fetch(b):
            pltpu.make_async_copy(
                tab_hbm.at[idx_all.at[pl.ds(0, 2 * _T)]], rows[b], gsem[b]
            ).wait()
            pltpu.make_async_copy(x_hbm.at[pl.ds(0, _T)], xbuf[b], xsem[b]).wait()

        def store(ci, b):
            pltpu.async_copy(xbuf[b], out_hbm.at[pl.ds(tok0 + ci * _T, _T)], osem[b])

        def wait_store(b):
            pltpu.make_async_copy(xbuf[b], out_hbm.at[pl.ds(0, _T)], osem[b]).wait()

        def compute(b):
            rv, xv = rows[b], xbuf[b]

            def tok(i, _):
                for h in range(2):
                    r = 2 * i + h
                    xo = h * _D
                    for j in range(_H // _L):
                        o1 = _L * j
                        o2 = _H + _L * j
                        a = xv[i, pl.ds(xo + o1, _L)]
                        bb = xv[i, pl.ds(xo + o2, _L)]
                        c = rv[r, pl.ds(o1, _L)]
                        s = rv[r, pl.ds(o2, _L)]
                        xv[i, pl.ds(xo + o1, _L)] = a * c - bb * s
                        xv[i, pl.ds(xo + o2, _L)] = bb * c + a * s
                return 0

            lax.fori_loop(0, _T, tok, 0, unroll=False)

        fetch(0, 0)
        fetch(1, 1)
        wait_fetch(0)
        compute(0)
        store(0, 0)

        def pair(pi, _):
            ci1 = 2 * pi + 1
            wait_store(0)
            fetch(ci1 + 1, 0)
            wait_fetch(1)
            compute(1)
            store(ci1, 1)
            ci2 = 2 * pi + 2
            wait_store(1)
            fetch(ci2 + 1, 1)
            wait_fetch(0)
            compute(0)
            store(ci2, 0)
            return 0

        lax.fori_loop(0, n_chunks // 2 - 1, pair, 0, unroll=False)

        wait_store(0)
        wait_fetch(1)
        compute(1)
        store(n_chunks - 1, 1)
        wait_store(1)

    return sc_rope


def _tc_body(buf_ref, x_ref, pm_ref, dt_ref, out_ref):
    del buf_ref
    dt = dt_ref[...]
    x = x_ref[...]
    pm = pm_ref[...]

    def one_axis(pv, xa):
        ang = pv.astype(jnp.float32) * dt
        c = jnp.cos(ang)
        s = jnp.sin(ang)
        a = xa[:, :_H]
        b = xa[:, _H:]
        return jnp.concatenate([a * c - b * s, b * c + a * s], axis=1)

    y1 = one_axis(pm[:, 0:1], x[:, :_D])
    y2 = one_axis(pm[:, 1:2], x[:, _D:])
    out_ref[...] = jnp.concatenate([y1, y2], axis=1)


def _tc_rope(buf, x, pm, n_sc: int):
    n = x.shape[0]
    n_tc = n - n_sc
    grid = (n_tc // _BT,)
    blk0 = n_sc // _BT

    return pl.pallas_call(
        _tc_body,
        grid=grid,
        in_specs=[
            pl.BlockSpec((_BT, _MODEL_DIM), lambda i: (blk0 + i, 0)),
            pl.BlockSpec((_BT, _MODEL_DIM), lambda i: (blk0 + i, 0)),
            pl.BlockSpec((_BT, 2), lambda i: (blk0 + i, 0)),
            pl.BlockSpec((1, _H), lambda i: (0, 0)),
        ],
        out_specs=pl.BlockSpec((_BT, _MODEL_DIM), lambda i: (blk0 + i, 0)),
        out_shape=jax.ShapeDtypeStruct((n, _MODEL_DIM), jnp.float32),
        input_output_aliases={0: 0},
    )(buf, x, pm, jnp.asarray(_div_term()))


def kernel(x, pos):
    b, sq, md = x.shape
    n = b * sq
    n_sc = (n * _SC_FRAC_NUM // _SC_FRAC_DEN) // (_NW * _T * 4) * (_NW * _T * 4)
    xf = x.reshape(n, md)
    pos = pos.astype(jnp.int32)
    pf = pos.reshape(2 * n)
    buf = _make_sc_rope(n, n_sc)(jnp.asarray(_fused_table()), xf, pf)
    out = _tc_rope(buf, xf, pos.reshape(n, 2), n_sc)
    return out.reshape(x.shape)


# speedup vs baseline: 1.4500x; 1.4500x over previous
import functools

import jax
import jax.numpy as jnp
import numpy as np
from jax import lax
from jax.experimental import pallas as pl
from jax.experimental.pallas import tpu as pltpu
from jax.experimental.pallas import tpu_sc as plsc

_MODEL_DIM = 1024
_MAX_POS = 1200
_TEMP = 10000.0
_D = _MODEL_DIM // 2
_H = _D // 2

_NC, _NS, _L = 2, 16, 16
_NW = _NC * _NS
_T = 16

_SC_FRAC_NUM, _SC_FRAC_DEN = 1, 2
_BT = 256
_PAD = 1280


@functools.lru_cache(maxsize=1)
def _fused_table():
    positions = np.arange(_MAX_POS, dtype=np.float64)[:, None]
    div_term = np.exp(np.arange(0, _D, 2, dtype=np.float64) * -(np.log(_TEMP) / _D))
    ang = positions * div_term
    tab = np.concatenate([np.cos(ang), np.sin(ang)], axis=-1)
    return tab.astype(np.float32)


@functools.lru_cache(maxsize=1)
def _tab_bf16_np():
    tab = np.zeros((_PAD, _D), dtype=np.float32)
    tab[:_MAX_POS] = _fused_table()
    return tab


def _tab_bf16():
    return jnp.asarray(_tab_bf16_np(), dtype=jnp.bfloat16)


def _make_sc_rope(n_sc: int):
    per_w = n_sc // _NW
    n_chunks = per_w // _T
    assert n_chunks % 2 == 0 and n_chunks >= 4
    mesh = plsc.VectorSubcoreMesh(core_axis_name="c", subcore_axis_name="s")

    @functools.partial(
        pl.kernel,
        mesh=mesh,
        out_type=jax.ShapeDtypeStruct((n_sc, _MODEL_DIM), jnp.float32),
        scratch_types=[
            pltpu.VMEM((2 * per_w,), jnp.int32),
            pltpu.VMEM((2 * _T, _D), jnp.float32),
            pltpu.VMEM((2 * _T, _D), jnp.float32),
            pltpu.VMEM((_T, _MODEL_DIM), jnp.float32),
            pltpu.VMEM((_T, _MODEL_DIM), jnp.float32),
            pltpu.SemaphoreType.DMA,
            pltpu.SemaphoreType.DMA,
            pltpu.SemaphoreType.DMA,
            pltpu.SemaphoreType.DMA,
            pltpu.SemaphoreType.DMA,
            pltpu.SemaphoreType.DMA,
        ],
    )
    def sc_rope(
        tab_hbm, x_hbm, pos_hbm, out_hbm,
        idx_all, rows0, rows1, x0, x1,
        gs0, gs1, xs0, xs1, os0, os1,
    ):
        rows = (rows0, rows1)
        xbuf = (x0, x1)
        gsem = (gs0, gs1)
        xsem = (xs0, xs1)
        osem = (os0, os1)
        wid = lax.axis_index("s") * _NC + lax.axis_index("c")
        tok0 = wid * per_w

        pltpu.sync_copy(pos_hbm.at[pl.ds(2 * tok0, 2 * per_w)], idx_all)

        def fetch(ci, b):
            pltpu.async_copy(
                tab_hbm.at[idx_all.at[pl.ds(ci * 2 * _T, 2 * _T)]], rows[b], gsem[b]
            )
            pltpu.async_copy(x_hbm.at[pl.ds(tok0 + ci * _T, _T)], xbuf[b], xsem[b])

        def wait_fetch(b):
            pltpu.make_async_copy(
                tab_hbm.at[idx_all.at[pl.ds(0, 2 * _T)]], rows[b], gsem[b]
            ).wait()
            pltpu.make_async_copy(x_hbm.at[pl.ds(0, _T)], xbuf[b], xsem[b]).wait()

        def store(ci, b):
            pltpu.async_copy(xbuf[b], out_hbm.at[pl.ds(tok0 + ci * _T, _T)], osem[b])

        def wait_store(b):
            pltpu.make_async_copy(xbuf[b], out_hbm.at[pl.ds(0, _T)], osem[b]).wait()

        def compute(b):
            rv, xv = rows[b], xbuf[b]

            def tok(i, _):
                for h in range(2):
                    r = 2 * i + h
                    xo = h * _D
                    for j in range(_H // _L):
                        o1 = _L * j
                        o2 = _H + _L * j
                        a = xv[i, pl.ds(xo + o1, _L)]
                        bb = xv[i, pl.ds(xo + o2, _L)]
                        c = rv[r, pl.ds(o1, _L)]
                        s = rv[r, pl.ds(o2, _L)]
                        xv[i, pl.ds(xo + o1, _L)] = a * c - bb * s
                        xv[i, pl.ds(xo + o2, _L)] = bb * c + a * s
                return 0

            lax.fori_loop(0, _T, tok, 0, unroll=False)

        fetch(0, 0)
        fetch(1, 1)
        wait_fetch(0)
        compute(0)
        store(0, 0)

        def pair(pi, _):
            ci1 = 2 * pi + 1
            wait_store(0)
            fetch(ci1 + 1, 0)
            wait_fetch(1)
            compute(1)
            store(ci1, 1)
            ci2 = 2 * pi + 2
            wait_store(1)
            fetch(ci2 + 1, 1)
            wait_fetch(0)
            compute(0)
            store(ci2, 0)
            return 0

        lax.fori_loop(0, n_chunks // 2 - 1, pair, 0, unroll=False)

        wait_store(0)
        wait_fetch(1)
        compute(1)
        store(n_chunks - 1, 1)
        wait_store(1)

    return sc_rope


def _tc_body(x_ref, pm_ref, tab_ref, out_ref):
    x = x_ref[...]
    pm = pm_ref[...]
    iota = lax.broadcasted_iota(jnp.int32, (1, _PAD), 1)

    def one_axis(pv, xa):
        oh = (pv == iota).astype(jnp.bfloat16)
        cs = jnp.dot(oh, tab_ref[...], preferred_element_type=jnp.float32)
        c = cs[:, :_H]
        s = cs[:, _H:]
        a = xa[:, :_H]
        b = xa[:, _H:]
        return jnp.concatenate([a * c - b * s, b * c + a * s], axis=1)

    y1 = one_axis(pm[:, 0:1], x[:, :_D])
    y2 = one_axis(pm[:, 1:2], x[:, _D:])
    out_ref[...] = jnp.concatenate([y1, y2], axis=1)


def _tc_rope(x, pm, n_sc: int):
    n = x.shape[0]
    n_tc = n - n_sc
    grid = (n_tc // _BT,)
    blk0 = n_sc // _BT

    return pl.pallas_call(
        _tc_body,
        grid=grid,
        in_specs=[
            pl.BlockSpec((_BT, _MODEL_DIM), lambda i: (blk0 + i, 0)),
            pl.BlockSpec((_BT, 2), lambda i: (blk0 + i, 0)),
            pl.BlockSpec((_PAD, _D), lambda i: (0, 0)),
        ],
        out_specs=pl.BlockSpec((_BT, _MODEL_DIM), lambda i: (blk0 + i, 0)),
        out_shape=jax.ShapeDtypeStruct((n, _MODEL_DIM), jnp.float32),
    )(x, pm, _tab_bf16())


def kernel(x, pos):
    b, sq, md = x.shape
    n = b * sq
    n_sc = (n * _SC_FRAC_NUM // _SC_FRAC_DEN) // (_NW * _T * 4) * (_NW * _T * 4)
    xf = x.reshape(n, md)
    pos = pos.astype(jnp.int32)
    pf = pos.reshape(2 * n)
    sc_out = _make_sc_rope(n_sc)(jnp.asarray(_fused_table()), xf, pf)
    tc_full = _tc_rope(xf, pos.reshape(n, 2), n_sc)
    out = lax.dynamic_update_slice(tc_full, sc_out, (0, 0))
    return out.reshape(x.shape)


# speedup vs baseline: 1.6050x; 1.1068x over previous
import functools

import jax
import jax.numpy as jnp
import numpy as np
from jax import lax
from jax.experimental import pallas as pl
from jax.experimental.pallas import tpu as pltpu
from jax.experimental.pallas import tpu_sc as plsc

_MODEL_DIM = 1024
_MAX_POS = 1200
_TEMP = 10000.0
_D = _MODEL_DIM // 2
_H = _D // 2

_NC, _NS, _L = 2, 16, 16
_NW = _NC * _NS
_T = 16

_SC_FRAC_NUM, _SC_FRAC_DEN = 7, 16
_BT = 256
_PAD = 1280


@functools.lru_cache(maxsize=1)
def _fused_table():
    positions = np.arange(_MAX_POS, dtype=np.float64)[:, None]
    div_term = np.exp(np.arange(0, _D, 2, dtype=np.float64) * -(np.log(_TEMP) / _D))
    ang = positions * div_term
    tab = np.concatenate([np.cos(ang), np.sin(ang)], axis=-1)
    return tab.astype(np.float32)


@functools.lru_cache(maxsize=1)
def _tab_bf16_np():
    tab = np.zeros((_PAD, _D), dtype=np.float32)
    tab[:_MAX_POS] = _fused_table()
    return tab


def _tab_bf16():
    return jnp.asarray(_tab_bf16_np(), dtype=jnp.bfloat16)


def _make_sc_rope(n_sc: int):
    per_w = n_sc // _NW
    n_chunks = per_w // _T
    assert n_chunks % 2 == 0 and n_chunks >= 4
    mesh = plsc.VectorSubcoreMesh(core_axis_name="c", subcore_axis_name="s")

    @functools.partial(
        pl.kernel,
        mesh=mesh,
        out_type=jax.ShapeDtypeStruct((n_sc, _MODEL_DIM), jnp.float32),
        scratch_types=[
            pltpu.VMEM((2 * per_w,), jnp.int32),
            pltpu.VMEM((2 * _T, _D), jnp.float32),
            pltpu.VMEM((2 * _T, _D), jnp.float32),
            pltpu.VMEM((_T, _MODEL_DIM), jnp.float32),
            pltpu.VMEM((_T, _MODEL_DIM), jnp.float32),
            pltpu.SemaphoreType.DMA,
            pltpu.SemaphoreType.DMA,
            pltpu.SemaphoreType.DMA,
            pltpu.SemaphoreType.DMA,
            pltpu.SemaphoreType.DMA,
            pltpu.SemaphoreType.DMA,
        ],
    )
    def sc_rope(
        tab_hbm, x_hbm, pos_hbm, out_hbm,
        idx_all, rows0, rows1, x0, x1,
        gs0, gs1, xs0, xs1, os0, os1,
    ):
        rows = (rows0, rows1)
        xbuf = (x0, x1)
        gsem = (gs0, gs1)
        xsem = (xs0, xs1)
        osem = (os0, os1)
        wid = lax.axis_index("s") * _NC + lax.axis_index("c")
        tok0 = wid * per_w

        pltpu.sync_copy(pos_hbm.at[pl.ds(2 * tok0, 2 * per_w)], idx_all)

        def fetch(ci, b):
            pltpu.async_copy(
                tab_hbm.at[idx_all.at[pl.ds(ci * 2 * _T, 2 * _T)]], rows[b], gsem[b]
            )
            pltpu.async_copy(x_hbm.at[pl.ds(tok0 + ci * _T, _T)], xbuf[b], xsem[b])

        def wait_fetch(b):
            pltpu.make_async_copy(
                tab_hbm.at[idx_all.at[pl.ds(0, 2 * _T)]], rows[b], gsem[b]
            ).wait()
            pltpu.make_async_copy(x_hbm.at[pl.ds(0, _T)], xbuf[b], xsem[b]).wait()

        def store(ci, b):
            pltpu.async_copy(xbuf[b], out_hbm.at[pl.ds(tok0 + ci * _T, _T)], osem[b])

        def wait_store(b):
            pltpu.make_async_copy(xbuf[b], out_hbm.at[pl.ds(0, _T)], osem[b]).wait()

        def compute(b):
            rv, xv = rows[b], xbuf[b]

            def tok(i, _):
                for h in range(2):
                    r = 2 * i + h
                    xo = h * _D
                    for j in range(_H // _L):
                        o1 = _L * j
                        o2 = _H + _L * j
                        a = xv[i, pl.ds(xo + o1, _L)]
                        bb = xv[i, pl.ds(xo + o2, _L)]
                        c = rv[r, pl.ds(o1, _L)]
                        s = rv[r, pl.ds(o2, _L)]
                        xv[i, pl.ds(xo + o1, _L)] = a * c - bb * s
                        xv[i, pl.ds(xo + o2, _L)] = bb * c + a * s
                return 0

            lax.fori_loop(0, _T, tok, 0, unroll=False)

        fetch(0, 0)
        fetch(1, 1)
        wait_fetch(0)
        compute(0)
        store(0, 0)

        def pair(pi, _):
            ci1 = 2 * pi + 1
            wait_store(0)
            fetch(ci1 + 1, 0)
            wait_fetch(1)
            compute(1)
            store(ci1, 1)
            ci2 = 2 * pi + 2
            wait_store(1)
            fetch(ci2 + 1, 1)
            wait_fetch(0)
            compute(0)
            store(ci2, 0)
            return 0

        lax.fori_loop(0, n_chunks // 2 - 1, pair, 0, unroll=False)

        wait_store(0)
        wait_fetch(1)
        compute(1)
        store(n_chunks - 1, 1)
        wait_store(1)

    return sc_rope


def _tc_body(x_ref, pm_ref, tab_ref, out_ref):
    x = x_ref[...]
    pm = pm_ref[...]
    iota = lax.broadcasted_iota(jnp.int32, (1, _PAD), 1)

    def one_axis(pv, xa):
        oh = (pv == iota).astype(jnp.bfloat16)
        cs = jnp.dot(oh, tab_ref[...], preferred_element_type=jnp.float32)
        c = cs[:, :_H]
        s = cs[:, _H:]
        a = xa[:, :_H]
        b = xa[:, _H:]
        return jnp.concatenate([a * c - b * s, b * c + a * s], axis=1)

    y1 = one_axis(pm[:, 0:1], x[:, :_D])
    y2 = one_axis(pm[:, 1:2], x[:, _D:])
    out_ref[...] = jnp.concatenate([y1, y2], axis=1)


def _tc_rope(x, pm, n_sc: int):
    n = x.shape[0]
    n_tc = n - n_sc
    grid = (n_tc // _BT,)
    blk0 = n_sc // _BT

    return pl.pallas_call(
        _tc_body,
        grid=grid,
        in_specs=[
            pl.BlockSpec((_BT, _MODEL_DIM), lambda i: (blk0 + i, 0)),
            pl.BlockSpec((_BT, 2), lambda i: (blk0 + i, 0)),
            pl.BlockSpec((_PAD, _D), lambda i: (0, 0)),
        ],
        out_specs=pl.BlockSpec((_BT, _MODEL_DIM), lambda i: (blk0 + i, 0)),
        out_shape=jax.ShapeDtypeStruct((n, _MODEL_DIM), jnp.float32),
    )(x, pm, _tab_bf16())


def kernel(x, pos):
    b, sq, md = x.shape
    n = b * sq
    n_sc = (n * _SC_FRAC_NUM // _SC_FRAC_DEN) // (_NW * _T * 4) * (_NW * _T * 4)
    xf = x.reshape(n, md)
    pos = pos.astype(jnp.int32)
    pf = pos.reshape(2 * n)
    sc_out = _make_sc_rope(n_sc)(jnp.asarray(_fused_table()), xf, pf)
    tc_full = _tc_rope(xf, pos.reshape(n, 2), n_sc)
    out = lax.dynamic_update_slice(tc_full, sc_out, (0, 0))
    return out.reshape(x.shape)


# speedup vs baseline: 1.6115x; 1.0041x over previous
import functools

import jax
import jax.numpy as jnp
import numpy as np
from jax import lax
from jax.experimental import pallas as pl
from jax.experimental.pallas import tpu as pltpu
from jax.experimental.pallas import tpu_sc as plsc

_MODEL_DIM = 1024
_MAX_POS = 1200
_TEMP = 10000.0
_D = _MODEL_DIM // 2
_H = _D // 2

_NC, _NS, _L = 2, 16, 16
_NW = _NC * _NS
_T = 16

_SC_FRAC_NUM, _SC_FRAC_DEN = 13, 32
_BT = 256
_PAD = 1280


@functools.lru_cache(maxsize=1)
def _fused_table():
    positions = np.arange(_MAX_POS, dtype=np.float64)[:, None]
    div_term = np.exp(np.arange(0, _D, 2, dtype=np.float64) * -(np.log(_TEMP) / _D))
    ang = positions * div_term
    tab = np.concatenate([np.cos(ang), np.sin(ang)], axis=-1)
    return tab.astype(np.float32)


@functools.lru_cache(maxsize=1)
def _tab_bf16_np():
    tab = np.zeros((_PAD, _D), dtype=np.float32)
    tab[:_MAX_POS] = _fused_table()
    return tab


def _tab_bf16():
    return jnp.asarray(_tab_bf16_np(), dtype=jnp.bfloat16)


def _make_sc_rope(n_sc: int):
    per_w = n_sc // _NW
    n_chunks = per_w // _T
    assert n_chunks % 2 == 0 and n_chunks >= 4
    mesh = plsc.VectorSubcoreMesh(core_axis_name="c", subcore_axis_name="s")

    @functools.partial(
        pl.kernel,
        mesh=mesh,
        out_type=jax.ShapeDtypeStruct((n_sc, _MODEL_DIM), jnp.float32),
        scratch_types=[
            pltpu.VMEM((2 * per_w,), jnp.int32),
            pltpu.VMEM((2 * _T, _D), jnp.float32),
            pltpu.VMEM((2 * _T, _D), jnp.float32),
            pltpu.VMEM((_T, _MODEL_DIM), jnp.float32),
            pltpu.VMEM((_T, _MODEL_DIM), jnp.float32),
            pltpu.SemaphoreType.DMA,
            pltpu.SemaphoreType.DMA,
            pltpu.SemaphoreType.DMA,
            pltpu.SemaphoreType.DMA,
            pltpu.SemaphoreType.DMA,
            pltpu.SemaphoreType.DMA,
        ],
    )
    def sc_rope(
        tab_hbm, x_hbm, pos_hbm, out_hbm,
        idx_all, rows0, rows1, x0, x1,
        gs0, gs1, xs0, xs1, os0, os1,
    ):
        rows = (rows0, rows1)
        xbuf = (x0, x1)
        gsem = (gs0, gs1)
        xsem = (xs0, xs1)
        osem = (os0, os1)
        wid = lax.axis_index("s") * _NC + lax.axis_index("c")
        tok0 = wid * per_w

        pltpu.sync_copy(pos_hbm.at[pl.ds(2 * tok0, 2 * per_w)], idx_all)

        def fetch(ci, b):
            pltpu.async_copy(
                tab_hbm.at[idx_all.at[pl.ds(ci * 2 * _T, 2 * _T)]], rows[b], gsem[b]
            )
            pltpu.async_copy(x_hbm.at[pl.ds(tok0 + ci * _T, _T)], xbuf[b], xsem[b])

        def wait_fetch(b):
            pltpu.make_async_copy(
                tab_hbm.at[idx_all.at[pl.ds(0, 2 * _T)]], rows[b], gsem[b]
            ).wait()
            pltpu.make_async_copy(x_hbm.at[pl.ds(0, _T)], xbuf[b], xsem[b]).wait()

        def store(ci, b):
            pltpu.async_copy(xbuf[b], out_hbm.at[pl.ds(tok0 + ci * _T, _T)], osem[b])

        def wait_store(b):
            pltpu.make_async_copy(xbuf[b], out_hbm.at[pl.ds(0, _T)], osem[b]).wait()

        def compute(b):
            rv, xv = rows[b], xbuf[b]

            def tok(i, _):
                for h in range(2):
                    r = 2 * i + h
                    xo = h * _D
                    for j in range(_H // _L):
                        o1 = _L * j
                        o2 = _H + _L * j
                        a = xv[i, pl.ds(xo + o1, _L)]
                        bb = xv[i, pl.ds(xo + o2, _L)]
                        c = rv[r, pl.ds(o1, _L)]
                        s = rv[r, pl.ds(o2, _L)]
                        xv[i, pl.ds(xo + o1, _L)] = a * c - bb * s
                        xv[i, pl.ds(xo + o2, _L)] = bb * c + a * s
                return 0

            lax.fori_loop(0, _T, tok, 0, unroll=False)

        fetch(0, 0)
        fetch(1, 1)
        wait_fetch(0)
        compute(0)
        store(0, 0)

        def pair(pi, _):
            ci1 = 2 * pi + 1
            wait_store(0)
            fetch(ci1 + 1, 0)
            wait_fetch(1)
            compute(1)
            store(ci1, 1)
            ci2 = 2 * pi + 2
            wait_store(1)
            fetch(ci2 + 1, 1)
            wait_fetch(0)
            compute(0)
            store(ci2, 0)
            return 0

        lax.fori_loop(0, n_chunks // 2 - 1, pair, 0, unroll=False)

        wait_store(0)
        wait_fetch(1)
        compute(1)
        store(n_chunks - 1, 1)
        wait_store(1)

    return sc_rope


def _tc_body(x_ref, pm_ref, tab_ref, out_ref):
    x = x_ref[...]
    pm = pm_ref[...]
    iota = lax.broadcasted_iota(jnp.int32, (1, _PAD), 1)

    def one_axis(pv, xa):
        oh = (pv == iota).astype(jnp.bfloat16)
        cs = jnp.dot(oh, tab_ref[...], preferred_element_type=jnp.float32)
        c = cs[:, :_H]
        s = cs[:, _H:]
        a = xa[:, :_H]
        b = xa[:, _H:]
        return jnp.concatenate([a * c - b * s, b * c + a * s], axis=1)

    y1 = one_axis(pm[:, 0:1], x[:, :_D])
    y2 = one_axis(pm[:, 1:2], x[:, _D:])
    out_ref[...] = jnp.concatenate([y1, y2], axis=1)


def _tc_rope(x, pm, n_sc: int):
    n = x.shape[0]
    n_tc = n - n_sc
    grid = (n_tc // _BT,)
    blk0 = n_sc // _BT

    return pl.pallas_call(
        _tc_body,
        grid=grid,
        in_specs=[
            pl.BlockSpec((_BT, _MODEL_DIM), lambda i: (blk0 + i, 0)),
            pl.BlockSpec((_BT, 2), lambda i: (blk0 + i, 0)),
            pl.BlockSpec((_PAD, _D), lambda i: (0, 0)),
        ],
        out_specs=pl.BlockSpec((_BT, _MODEL_DIM), lambda i: (blk0 + i, 0)),
        out_shape=jax.ShapeDtypeStruct((n, _MODEL_DIM), jnp.float32),
    )(x, pm, _tab_bf16())


def kernel(x, pos):
    b, sq, md = x.shape
    n = b * sq
    n_sc = (n * _SC_FRAC_NUM // _SC_FRAC_DEN) // (_NW * _T * 4) * (_NW * _T * 4)
    xf = x.reshape(n, md)
    pos = pos.astype(jnp.int32)
    pf = pos.reshape(2 * n)
    sc_out = _make_sc_rope(n_sc)(jnp.asarray(_fused_table()), xf, pf)
    tc_full = _tc_rope(xf, pos.reshape(n, 2), n_sc)
    out = lax.dynamic_update_slice(tc_full, sc_out, (0, 0))
    return out.reshape(x.shape)


# speedup vs baseline: 1.7708x; 1.0989x over previous
import functools

import jax
import jax.numpy as jnp
import numpy as np
from jax import lax
from jax.experimental import pallas as pl
from jax.experimental.pallas import tpu as pltpu
from jax.experimental.pallas import tpu_sc as plsc

_MODEL_DIM = 1024
_MAX_POS = 1200
_TEMP = 10000.0
_D = _MODEL_DIM // 2
_H = _D // 2

_NC, _NS, _L = 2, 16, 16
_NW = _NC * _NS
_T = 16

_SC_FRAC_NUM, _SC_FRAC_DEN = 13, 32
_BT = 512
_PAD = 1280


@functools.lru_cache(maxsize=1)
def _fused_table():
    positions = np.arange(_MAX_POS, dtype=np.float64)[:, None]
    div_term = np.exp(np.arange(0, _D, 2, dtype=np.float64) * -(np.log(_TEMP) / _D))
    ang = positions * div_term
    tab = np.concatenate([np.cos(ang), np.sin(ang)], axis=-1)
    return tab.astype(np.float32)


@functools.lru_cache(maxsize=1)
def _tab_bf16_np():
    tab = np.zeros((_PAD, _D), dtype=np.float32)
    tab[:_MAX_POS] = _fused_table()
    return tab


def _tab_bf16():
    return jnp.asarray(_tab_bf16_np(), dtype=jnp.bfloat16)


def _make_sc_rope(n_sc: int):
    per_w = n_sc // _NW
    n_chunks = per_w // _T
    assert n_chunks % 2 == 0 and n_chunks >= 4
    mesh = plsc.VectorSubcoreMesh(core_axis_name="c", subcore_axis_name="s")

    @functools.partial(
        pl.kernel,
        mesh=mesh,
        out_type=jax.ShapeDtypeStruct((n_sc, _MODEL_DIM), jnp.float32),
        scratch_types=[
            pltpu.VMEM((2 * per_w,), jnp.int32),
            pltpu.VMEM((2 * _T, _D), jnp.float32),
            pltpu.VMEM((2 * _T, _D), jnp.float32),
            pltpu.VMEM((_T, _MODEL_DIM), jnp.float32),
            pltpu.VMEM((_T, _MODEL_DIM), jnp.float32),
            pltpu.SemaphoreType.DMA,
            pltpu.SemaphoreType.DMA,
            pltpu.SemaphoreType.DMA,
            pltpu.SemaphoreType.DMA,
            pltpu.SemaphoreType.DMA,
            pltpu.SemaphoreType.DMA,
        ],
    )
    def sc_rope(
        tab_hbm, x_hbm, pos_hbm, out_hbm,
        idx_all, rows0, rows1, x0, x1,
        gs0, gs1, xs0, xs1, os0, os1,
    ):
        rows = (rows0, rows1)
        xbuf = (x0, x1)
        gsem = (gs0, gs1)
        xsem = (xs0, xs1)
        osem = (os0, os1)
        wid = lax.axis_index("s") * _NC + lax.axis_index("c")
        tok0 = wid * per_w

        pltpu.sync_copy(pos_hbm.at[pl.ds(2 * tok0, 2 * per_w)], idx_all)

        def fetch(ci, b):
            pltpu.async_copy(
                tab_hbm.at[idx_all.at[pl.ds(ci * 2 * _T, 2 * _T)]], rows[b], gsem[b]
            )
            pltpu.async_copy(x_hbm.at[pl.ds(tok0 + ci * _T, _T)], xbuf[b], xsem[b])

        def wait_fetch(b):
            pltpu.make_async_copy(
                tab_hbm.at[idx_all.at[pl.ds(0, 2 * _T)]], rows[b], gsem[b]
            ).wait()
            pltpu.make_async_copy(x_hbm.at[pl.ds(0, _T)], xbuf[b], xsem[b]).wait()

        def store(ci, b):
            pltpu.async_copy(xbuf[b], out_hbm.at[pl.ds(tok0 + ci * _T, _T)], osem[b])

        def wait_store(b):
            pltpu.make_async_copy(xbuf[b], out_hbm.at[pl.ds(0, _T)], osem[b]).wait()

        def compute(b):
            rv, xv = rows[b], xbuf[b]

            def tok(i, _):
                for h in range(2):
                    r = 2 * i + h
                    xo = h * _D
                    for j in range(_H // _L):
                        o1 = _L * j
                        o2 = _H + _L * j
                        a = xv[i, pl.ds(xo + o1, _L)]
                        bb = xv[i, pl.ds(xo + o2, _L)]
                        c = rv[r, pl.ds(o1, _L)]
                        s = rv[r, pl.ds(o2, _L)]
                        xv[i, pl.ds(xo + o1, _L)] = a * c - bb * s
                        xv[i, pl.ds(xo + o2, _L)] = bb * c + a * s
                return 0

            lax.fori_loop(0, _T, tok, 0, unroll=False)

        fetch(0, 0)
        fetch(1, 1)
        wait_fetch(0)
        compute(0)
        store(0, 0)

        def pair(pi, _):
            ci1 = 2 * pi + 1
            wait_store(0)
            fetch(ci1 + 1, 0)
            wait_fetch(1)
            compute(1)
            store(ci1, 1)
            ci2 = 2 * pi + 2
            wait_store(1)
            fetch(ci2 + 1, 1)
            wait_fetch(0)
            compute(0)
            store(ci2, 0)
            return 0

        lax.fori_loop(0, n_chunks // 2 - 1, pair, 0, unroll=False)

        wait_store(0)
        wait_fetch(1)
        compute(1)
        store(n_chunks - 1, 1)
        wait_store(1)

    return sc_rope


def _tc_body(x_ref, pm_ref, tab_ref, out_ref):
    x = x_ref[...]
    pm = pm_ref[...]
    iota = lax.broadcasted_iota(jnp.int32, (1, _PAD), 1)

    def one_axis(pv, xa):
        oh = (pv == iota).astype(jnp.bfloat16)
        cs = jnp.dot(oh, tab_ref[...], preferred_element_type=jnp.float32)
        c = cs[:, :_H]
        s = cs[:, _H:]
        a = xa[:, :_H]
        b = xa[:, _H:]
        return jnp.concatenate([a * c - b * s, b * c + a * s], axis=1)

    y1 = one_axis(pm[:, 0:1], x[:, :_D])
    y2 = one_axis(pm[:, 1:2], x[:, _D:])
    out_ref[...] = jnp.concatenate([y1, y2], axis=1)


def _tc_rope(x, pm, n_sc: int):
    n = x.shape[0]
    n_tc = n - n_sc
    grid = (n_tc // _BT,)
    blk0 = n_sc // _BT

    return pl.pallas_call(
        _tc_body,
        grid=grid,
        in_specs=[
            pl.BlockSpec((_BT, _MODEL_DIM), lambda i: (blk0 + i, 0)),
            pl.BlockSpec((_BT, 2), lambda i: (blk0 + i, 0)),
            pl.BlockSpec((_PAD, _D), lambda i: (0, 0)),
        ],
        out_specs=pl.BlockSpec((_BT, _MODEL_DIM), lambda i: (blk0 + i, 0)),
        out_shape=jax.ShapeDtypeStruct((n, _MODEL_DIM), jnp.float32),
    )(x, pm, _tab_bf16())


def kernel(x, pos):
    b, sq, md = x.shape
    n = b * sq
    n_sc = (n * _SC_FRAC_NUM // _SC_FRAC_DEN) // (_NW * _T * 4) * (_NW * _T * 4)
    xf = x.reshape(n, md)
    pos = pos.astype(jnp.int32)
    pf = pos.reshape(2 * n)
    sc_out = _make_sc_rope(n_sc)(jnp.asarray(_fused_table()), xf, pf)
    tc_full = _tc_rope(xf, pos.reshape(n, 2), n_sc)
    out = lax.dynamic_update_slice(tc_full, sc_out, (0, 0))
    return out.reshape(x.shape)


# speedup vs baseline: 1.8054x; 1.0196x over previous
import functools

import jax
import jax.numpy as jnp
import numpy as np
from jax import lax
from jax.experimental import pallas as pl
from jax.experimental.pallas import tpu as pltpu
from jax.experimental.pallas import tpu_sc as plsc

_MODEL_DIM = 1024
_MAX_POS = 1200
_TEMP = 10000.0
_D = _MODEL_DIM // 2
_H = _D // 2

_NC, _NS, _L = 2, 16, 16
_NW = _NC * _NS
_T = 16

_SC_FRAC_NUM, _SC_FRAC_DEN = 11, 32
_BT = 512
_PAD = 1280


@functools.lru_cache(maxsize=1)
def _fused_table():
    positions = np.arange(_MAX_POS, dtype=np.float64)[:, None]
    div_term = np.exp(np.arange(0, _D, 2, dtype=np.float64) * -(np.log(_TEMP) / _D))
    ang = positions * div_term
    tab = np.concatenate([np.cos(ang), np.sin(ang)], axis=-1)
    return tab.astype(np.float32)


@functools.lru_cache(maxsize=1)
def _tab_bf16_np():
    tab = np.zeros((_PAD, _D), dtype=np.float32)
    tab[:_MAX_POS] = _fused_table()
    return tab


def _tab_bf16():
    return jnp.asarray(_tab_bf16_np(), dtype=jnp.bfloat16)


def _make_sc_rope(n_sc: int):
    per_w = n_sc // _NW
    n_chunks = per_w // _T
    assert n_chunks % 2 == 0 and n_chunks >= 4
    mesh = plsc.VectorSubcoreMesh(core_axis_name="c", subcore_axis_name="s")

    @functools.partial(
        pl.kernel,
        mesh=mesh,
        out_type=jax.ShapeDtypeStruct((n_sc, _MODEL_DIM), jnp.float32),
        scratch_types=[
            pltpu.VMEM((2 * per_w,), jnp.int32),
            pltpu.VMEM((2 * _T, _D), jnp.float32),
            pltpu.VMEM((2 * _T, _D), jnp.float32),
            pltpu.VMEM((_T, _MODEL_DIM), jnp.float32),
            pltpu.VMEM((_T, _MODEL_DIM), jnp.float32),
            pltpu.SemaphoreType.DMA,
            pltpu.SemaphoreType.DMA,
            pltpu.SemaphoreType.DMA,
            pltpu.SemaphoreType.DMA,
            pltpu.SemaphoreType.DMA,
            pltpu.SemaphoreType.DMA,
        ],
    )
    def sc_rope(
        tab_hbm, x_hbm, pos_hbm, out_hbm,
        idx_all, rows0, rows1, x0, x1,
        gs0, gs1, xs0, xs1, os0, os1,
    ):
        rows = (rows0, rows1)
        xbuf = (x0, x1)
        gsem = (gs0, gs1)
        xsem = (xs0, xs1)
        osem = (os0, os1)
        wid = lax.axis_index("s") * _NC + lax.axis_index("c")
        tok0 = wid * per_w

        pltpu.sync_copy(pos_hbm.at[pl.ds(2 * tok0, 2 * per_w)], idx_all)

        def fetch(ci, b):
            pltpu.async_copy(
                tab_hbm.at[idx_all.at[pl.ds(ci * 2 * _T, 2 * _T)]], rows[b], gsem[b]
            )
            pltpu.async_copy(x_hbm.at[pl.ds(tok0 + ci * _T, _T)], xbuf[b], xsem[b])

        def wait_fetch(b):
            pltpu.make_async_copy(
                tab_hbm.at[idx_all.at[pl.ds(0, 2 * _T)]], rows[b], gsem[b]
            ).wait()
            pltpu.make_async_copy(x_hbm.at[pl.ds(0, _T)], xbuf[b], xsem[b]).wait()

        def store(ci, b):
            pltpu.async_copy(xbuf[b], out_hbm.at[pl.ds(tok0 + ci * _T, _T)], osem[b])

        def wait_store(b):
            pltpu.make_async_copy(xbuf[b], out_hbm.at[pl.ds(0, _T)], osem[b]).wait()

        def compute(b):
            rv, xv = rows[b], xbuf[b]

            def tok(i, _):
                for h in range(2):
                    r = 2 * i + h
                    xo = h * _D
                    for j in range(_H // _L):
                        o1 = _L * j
                        o2 = _H + _L * j
                        a = xv[i, pl.ds(xo + o1, _L)]
                        bb = xv[i, pl.ds(xo + o2, _L)]
                        c = rv[r, pl.ds(o1, _L)]
                        s = rv[r, pl.ds(o2, _L)]
                        xv[i, pl.ds(xo + o1, _L)] = a * c - bb * s
                        xv[i, pl.ds(xo + o2, _L)] = bb * c + a * s
                return 0

            lax.fori_loop(0, _T, tok, 0, unroll=False)

        fetch(0, 0)
        fetch(1, 1)
        wait_fetch(0)
        compute(0)
        store(0, 0)

        def pair(pi, _):
            ci1 = 2 * pi + 1
            wait_store(0)
            fetch(ci1 + 1, 0)
            wait_fetch(1)
            compute(1)
            store(ci1, 1)
            ci2 = 2 * pi + 2
            wait_store(1)
            fetch(ci2 + 1, 1)
            wait_fetch(0)
            compute(0)
            store(ci2, 0)
            return 0

        lax.fori_loop(0, n_chunks // 2 - 1, pair, 0, unroll=False)

        wait_store(0)
        wait_fetch(1)
        compute(1)
        store(n_chunks - 1, 1)
        wait_store(1)

    return sc_rope


def _tc_body(x_ref, pm_ref, tab_ref, out_ref):
    x = x_ref[...]
    pm = pm_ref[...]
    iota = lax.broadcasted_iota(jnp.int32, (1, _PAD), 1)

    def one_axis(pv, xa):
        oh = (pv == iota).astype(jnp.bfloat16)
        cs = jnp.dot(oh, tab_ref[...], preferred_element_type=jnp.float32)
        c = cs[:, :_H]
        s = cs[:, _H:]
        a = xa[:, :_H]
        b = xa[:, _H:]
        return jnp.concatenate([a * c - b * s, b * c + a * s], axis=1)

    y1 = one_axis(pm[:, 0:1], x[:, :_D])
    y2 = one_axis(pm[:, 1:2], x[:, _D:])
    out_ref[...] = jnp.concatenate([y1, y2], axis=1)


def _tc_rope(x, pm, n_sc: int):
    n = x.shape[0]
    n_tc = n - n_sc
    grid = (n_tc // _BT,)
    blk0 = n_sc // _BT

    return pl.pallas_call(
        _tc_body,
        grid=grid,
        in_specs=[
            pl.BlockSpec((_BT, _MODEL_DIM), lambda i: (blk0 + i, 0)),
            pl.BlockSpec((_BT, 2), lambda i: (blk0 + i, 0)),
            pl.BlockSpec((_PAD, _D), lambda i: (0, 0)),
        ],
        out_specs=pl.BlockSpec((_BT, _MODEL_DIM), lambda i: (blk0 + i, 0)),
        out_shape=jax.ShapeDtypeStruct((n, _MODEL_DIM), jnp.float32),
    )(x, pm, _tab_bf16())


def kernel(x, pos):
    b, sq, md = x.shape
    n = b * sq
    n_sc = (n * _SC_FRAC_NUM // _SC_FRAC_DEN) // (_NW * _T * 2) * (_NW * _T * 2)
    xf = x.reshape(n, md)
    pos = pos.astype(jnp.int32)
    pf = pos.reshape(2 * n)
    sc_out = _make_sc_rope(n_sc)(jnp.asarray(_fused_table()), xf, pf)
    tc_full = _tc_rope(xf, pos.reshape(n, 2), n_sc)
    out = lax.dynamic_update_slice(tc_full, sc_out, (0, 0))
    return out.reshape(x.shape)
